# baseline (speedup 1.0000x reference)
import functools

import numpy as np
import jax
import jax.numpy as jnp
from jax import lax
from jax.experimental import pallas as pl
from jax.experimental.pallas import tpu as pltpu
from jax.experimental.pallas import tpu_sc as plsc

NUM_LEVELS = 16
LEVEL_DIM = 2
P1 = int(np.uint32(2654435761).view(np.int32))
P2 = 805459861
HASH_MASK = (1 << 19) - 1
HASH_N = 1 << 19

_OFFS = [0, 4096, 36864, 299008]
for _ in range(13):
    _OFFS.append(_OFFS[-1] + HASH_N)

NC, NS = 2, 16
NW = NC * NS
CH = 128
NG = CH // 16


def _encode_body(
    spw, nch, inp, emb0, emb1, out,
    xyz, lvlout, idxA, idxB, r0A, r1A, r0B, r1B, sp, semA, semB,
):
    cid = lax.axis_index("c")
    sid = lax.axis_index("s")
    wid = sid * NC + cid
    wbase = wid * spw

    def corner_w(wa, wbv, c):
        w = wbv[0] if (c & 1) else wa[0]
        w = w * (wbv[1] if (c >> 1) & 1 else wa[1])
        w = w * (wbv[2] if (c >> 2) & 1 else wa[2])
        return w

    def smooth(g16, scale_f):
        li, wa, wbv = [], [], []
        for d in range(3):
            x01 = (xyz[d, pl.ds(g16, 16)] + 1.0) * 0.5
            pos = x01 * scale_f
            lid = pos.astype(jnp.int32)
            t = pos - lid.astype(jnp.float32)
            w = (t * t) * (3.0 - (t + t))
            li.append(lid)
            wbv.append(w)
            wa.append(1.0 - w)
        return li, wa, wbv

    pltpu.sync_copy(inp.at[:, pl.ds(wbase, spw)], xyz)

    def level_pass(lvl, nrows, idx_of_group):
        off = _OFFS[3] + (lvl - 3) * HASH_N if nrows is None else _OFFS[lvl]

        res = 16 << lvl
        scale_f = (res - 1).astype(jnp.float32) if nrows is None else float(res - 1)

        def compute_chunk(ci, idxb, rows0, rows1, sem):
            coff = ci * CH

            @pl.loop(0, NG)
            def _g(g):
                g16 = coff + g * 16
                li, _, _ = smooth(g16, scale_f)
                for c, iv in idx_of_group(li):
                    idxb[c, pl.ds(g * 16, 16)] = iv + off

            for c in range(8):
                pltpu.async_copy(emb0.at[idxb.at[c]], rows0.at[c], sem)
                pltpu.async_copy(emb1.at[idxb.at[c]], rows1.at[c], sem)

        def accum_chunk(ci, idxb, rows0, rows1, sem):
            coff = ci * CH
            for c in range(8):
                pltpu.make_async_copy(emb0.at[idxb.at[c]], rows0.at[c], sem).wait()
                pltpu.make_async_copy(emb1.at[idxb.at[c]], rows1.at[c], sem).wait()

            @pl.loop(0, NG)
            def _g(g):
                g16 = coff + g * 16
                _, wa, wbv = smooth(g16, scale_f)
                acc0 = jnp.zeros((16,), jnp.float32)
                acc1 = jnp.zeros((16,), jnp.float32)
                for c in range(8):
                    wv = corner_w(wa, wbv, c)
                    acc0 = acc0 + wv * rows0[c, pl.ds(g * 16, 16)]
                    acc1 = acc1 + wv * rows1[c, pl.ds(g * 16, 16)]
                lvlout[0, pl.ds(g16, 16)] = acc0
                lvlout[1, pl.ds(g16, 16)] = acc1

        compute_chunk(0, idxA, r0A, r1A, semA)

        @pl.loop(0, nch // 2)
        def _pair(p):
            even = p * 2
            compute_chunk(even + 1, idxB, r0B, r1B, semB)
            accum_chunk(even, idxA, r0A, r1A, semA)

            @pl.when(even + 2 < nch)
            def _():
                compute_chunk(even + 2, idxA, r0A, r1A, semA)

            accum_chunk(even + 1, idxB, r0B, r1B, semB)

        pltpu.sync_copy(lvlout, out.at[pl.ds(lvl * 2, 2), pl.ds(wbase, spw)])

    for lvl in range(3):
        res = 16 << lvl

        def dense_idx(li, res=res):
            s0 = li[0] + li[1] * res + li[2] * (res * res)
            for c in range(8):
                cc = (c & 1) + ((c >> 1) & 1) * res + ((c >> 2) & 1) * (res * res)
                yield c, s0 + cc

        level_pass(lvl, res ** 3, dense_idx)

    @pl.loop(3, NUM_LEVELS)
    def _lvl(lvl):
        def hash_idx(li):
            hx = (li[0], li[0] + 1)
            hy0 = li[1] * P1
            hy = (hy0, hy0 + P1)
            hz0 = li[2] * P2
            hz = (hz0, hz0 + P2)
            for c in range(8):
                h = hx[c & 1] ^ hy[(c >> 1) & 1] ^ hz[(c >> 2) & 1]
                yield c, h & HASH_MASK

        level_pass(lvl, None, hash_idx)


def kernel(inputs, embeddings):
    b = inputs.shape[0]
    assert b % (NW * CH * 2) == 0
    spw = b // NW
    nch = spw // CH
    mesh = plsc.VectorSubcoreMesh(
        core_axis_name="c", subcore_axis_name="s", num_cores=NC, num_subcores=NS
    )
    fn = pl.kernel(
        functools.partial(_encode_body, spw, nch),
        out_type=jax.ShapeDtypeStruct((NUM_LEVELS * LEVEL_DIM, b), jnp.float32),
        mesh=mesh,
        compiler_params=pltpu.CompilerParams(use_tc_tiling_on_sc=False),
        scratch_types=[
            pltpu.VMEM((3, spw), jnp.float32),
            pltpu.VMEM((2, spw), jnp.float32),
            pltpu.VMEM((8, CH), jnp.int32),
            pltpu.VMEM((8, CH), jnp.int32),
            pltpu.VMEM((8, CH), jnp.float32),
            pltpu.VMEM((8, CH), jnp.float32),
            pltpu.VMEM((8, CH), jnp.float32),
            pltpu.VMEM((8, CH), jnp.float32),
            pltpu.VMEM_SHARED((2, HASH_N), jnp.float32),
            pltpu.SemaphoreType.DMA,
            pltpu.SemaphoreType.DMA,
        ],
    )
    embp = embeddings.T
    return fn(inputs.T, embp[0], embp[1]).T


# speedup vs baseline: 3.0135x; 3.0135x over previous
import functools

import numpy as np
import jax
import jax.numpy as jnp
from jax import lax
from jax.experimental import pallas as pl
from jax.experimental.pallas import tpu as pltpu
from jax.experimental.pallas import tpu_sc as plsc

NUM_LEVELS = 16
LEVEL_DIM = 2
P1 = int(np.uint32(2654435761).view(np.int32))
P2 = 805459861
HASH_MASK = (1 << 19) - 1
HASH_N = 1 << 19

_OFFS = [0, 4096, 36864, 299008]
for _ in range(13):
    _OFFS.append(_OFFS[-1] + HASH_N)

NC, NS = 2, 16
NW = NC * NS
CH = 128
NG = CH // 16


def _encode_body(
    spw, nch, inp, emb0, emb1, out,
    xyz, lvlout, idxA, idxB, r0A, r1A, r0B, r1B, sp, semA, semB,
):
    cid = lax.axis_index("c")
    sid = lax.axis_index("s")
    wid = sid * NC + cid
    wbase = wid * spw

    def corner_w(wa, wbv, c):
        w = wbv[0] if (c & 1) else wa[0]
        w = w * (wbv[1] if (c >> 1) & 1 else wa[1])
        w = w * (wbv[2] if (c >> 2) & 1 else wa[2])
        return w

    def smooth(g16, scale_f):
        li, wa, wbv = [], [], []
        for d in range(3):
            x01 = (xyz[d, pl.ds(g16, 16)] + 1.0) * 0.5
            pos = x01 * scale_f
            lid = pos.astype(jnp.int32)
            t = pos - lid.astype(jnp.float32)
            w = (t * t) * (3.0 - (t + t))
            li.append(lid)
            wbv.append(w)
            wa.append(1.0 - w)
        return li, wa, wbv

    pltpu.sync_copy(inp.at[:, pl.ds(wbase, spw)], xyz)

    def level_pass(lvl, nrows, idx_of_group):
        @pl.when(sid == 0)
        def _():
            off = _OFFS[3] + (lvl - 3) * HASH_N if nrows is None else _OFFS[lvl]
            n = HASH_N if nrows is None else nrows
            pltpu.sync_copy(emb0.at[pl.ds(off, n)], sp.at[0, pl.ds(0, n)])
            pltpu.sync_copy(emb1.at[pl.ds(off, n)], sp.at[1, pl.ds(0, n)])

        plsc.subcore_barrier()

        res = 16 << lvl
        scale_f = (res - 1).astype(jnp.float32) if nrows is None else float(res - 1)

        def compute_chunk(ci, idxb, rows0, rows1, sem):
            coff = ci * CH

            @pl.loop(0, NG)
            def _g(g):
                g16 = coff + g * 16
                li, _, _ = smooth(g16, scale_f)
                for c, iv in idx_of_group(li):
                    idxb[c, pl.ds(g * 16, 16)] = iv

            for c in range(8):
                pltpu.async_copy(sp.at[0].at[idxb.at[c]], rows0.at[c], sem)
                pltpu.async_copy(sp.at[1].at[idxb.at[c]], rows1.at[c], sem)

        def accum_chunk(ci, idxb, rows0, rows1, sem):
            coff = ci * CH
            for c in range(8):
                pltpu.make_async_copy(sp.at[0].at[idxb.at[c]], rows0.at[c], sem).wait()
                pltpu.make_async_copy(sp.at[1].at[idxb.at[c]], rows1.at[c], sem).wait()

            @pl.loop(0, NG)
            def _g(g):
                g16 = coff + g * 16
                _, wa, wbv = smooth(g16, scale_f)
                acc0 = jnp.zeros((16,), jnp.float32)
                acc1 = jnp.zeros((16,), jnp.float32)
                for c in range(8):
                    wv = corner_w(wa, wbv, c)
                    acc0 = acc0 + wv * rows0[c, pl.ds(g * 16, 16)]
                    acc1 = acc1 + wv * rows1[c, pl.ds(g * 16, 16)]
                lvlout[0, pl.ds(g16, 16)] = acc0
                lvlout[1, pl.ds(g16, 16)] = acc1

        @pl.loop(0, nch)
        def _seq(ci):
            compute_chunk(ci, idxA, r0A, r1A, semA)
            accum_chunk(ci, idxA, r0A, r1A, semA)

        pltpu.sync_copy(lvlout, out.at[pl.ds(lvl * 2, 2), pl.ds(wbase, spw)])
        plsc.subcore_barrier()

    for lvl in range(3):
        res = 16 << lvl

        def dense_idx(li, res=res):
            s0 = li[0] + li[1] * res + li[2] * (res * res)
            for c in range(8):
                cc = (c & 1) + ((c >> 1) & 1) * res + ((c >> 2) & 1) * (res * res)
                yield c, s0 + cc

        level_pass(lvl, res ** 3, dense_idx)

    @pl.loop(3, NUM_LEVELS)
    def _lvl(lvl):
        def hash_idx(li):
            hx = (li[0], li[0] + 1)
            hy0 = li[1] * P1
            hy = (hy0, hy0 + P1)
            hz0 = li[2] * P2
            hz = (hz0, hz0 + P2)
            for c in range(8):
                h = hx[c & 1] ^ hy[(c >> 1) & 1] ^ hz[(c >> 2) & 1]
                yield c, h & HASH_MASK

        level_pass(lvl, None, hash_idx)


def kernel(inputs, embeddings):
    b = inputs.shape[0]
    assert b % (NW * CH * 2) == 0
    spw = b // NW
    nch = spw // CH
    mesh = plsc.VectorSubcoreMesh(
        core_axis_name="c", subcore_axis_name="s", num_cores=NC, num_subcores=NS
    )
    fn = pl.kernel(
        functools.partial(_encode_body, spw, nch),
        out_type=jax.ShapeDtypeStruct((NUM_LEVELS * LEVEL_DIM, b), jnp.float32),
        mesh=mesh,
        compiler_params=pltpu.CompilerParams(use_tc_tiling_on_sc=False),
        scratch_types=[
            pltpu.VMEM((3, spw), jnp.float32),
            pltpu.VMEM((2, spw), jnp.float32),
            pltpu.VMEM((8, CH), jnp.int32),
            pltpu.VMEM((8, CH), jnp.int32),
            pltpu.VMEM((8, CH), jnp.float32),
            pltpu.VMEM((8, CH), jnp.float32),
            pltpu.VMEM((8, CH), jnp.float32),
            pltpu.VMEM((8, CH), jnp.float32),
            pltpu.VMEM_SHARED((2, HASH_N), jnp.float32),
            pltpu.SemaphoreType.DMA,
            pltpu.SemaphoreType.DMA,
        ],
    )
    embp = embeddings.T
    return fn(inputs.T, embp[0], embp[1]).T


# speedup vs baseline: 3.2259x; 1.0705x over previous
import functools

import numpy as np
import jax
import jax.numpy as jnp
from jax import lax
from jax.experimental import pallas as pl
from jax.experimental.pallas import tpu as pltpu
from jax.experimental.pallas import tpu_sc as plsc

NUM_LEVELS = 16
LEVEL_DIM = 2
P1 = int(np.uint32(2654435761).view(np.int32))
P2 = 805459861
HASH_MASK = (1 << 19) - 1
HASH_N = 1 << 19

_OFFS = [0, 4096, 36864, 299008]
for _ in range(13):
    _OFFS.append(_OFFS[-1] + HASH_N)

NC, NS = 2, 16
NW = NC * NS
CH = 128
NG = CH // 16


def _encode_body(
    spw, nch, inp, emb0, emb1, out,
    xyz, lvlout, idxA, idxB, r0A, r1A, r0B, r1B, sp, semA, semB,
):
    cid = lax.axis_index("c")
    sid = lax.axis_index("s")
    wid = sid * NC + cid
    wbase = wid * spw

    def corner_w(wa, wbv, c):
        w = wbv[0] if (c & 1) else wa[0]
        w = w * (wbv[1] if (c >> 1) & 1 else wa[1])
        w = w * (wbv[2] if (c >> 2) & 1 else wa[2])
        return w

    def smooth(g16, scale_f):
        li, wa, wbv = [], [], []
        for d in range(3):
            x01 = (xyz[d, pl.ds(g16, 16)] + 1.0) * 0.5
            pos = x01 * scale_f
            lid = pos.astype(jnp.int32)
            t = pos - lid.astype(jnp.float32)
            w = (t * t) * (3.0 - (t + t))
            li.append(lid)
            wbv.append(w)
            wa.append(1.0 - w)
        return li, wa, wbv

    pltpu.sync_copy(inp.at[:, pl.ds(wbase, spw)], xyz)

    def level_pass(lvl, nrows, idx_of_group):
        @pl.when(sid == 0)
        def _():
            off = _OFFS[3] + (lvl - 3) * HASH_N if nrows is None else _OFFS[lvl]
            n = HASH_N if nrows is None else nrows
            pltpu.sync_copy(emb0.at[pl.ds(off, n)], sp.at[0, pl.ds(0, n)])
            pltpu.sync_copy(emb1.at[pl.ds(off, n)], sp.at[1, pl.ds(0, n)])

        plsc.subcore_barrier()

        res = 16 << lvl
        scale_f = (res - 1).astype(jnp.float32) if nrows is None else float(res - 1)

        def compute_chunk(ci, idxb, rows0, rows1, sem):
            coff = ci * CH

            @pl.loop(0, NG)
            def _g(g):
                g16 = coff + g * 16
                li, _, _ = smooth(g16, scale_f)
                for c, iv in idx_of_group(li):
                    idxb[c, pl.ds(g * 16, 16)] = iv

            for c in range(8):
                pltpu.async_copy(sp.at[0].at[idxb.at[c]], rows0.at[c], sem)
                pltpu.async_copy(sp.at[1].at[idxb.at[c]], rows1.at[c], sem)

        def weights_chunk(ci, wbuf):
            coff = ci * CH

            @pl.loop(0, NG)
            def _g(g):
                g16 = coff + g * 16
                _, wa, wbv = smooth(g16, scale_f)
                for c in range(8):
                    wbuf[c, pl.ds(g * 16, 16)] = corner_w(wa, wbv, c)

        def mac_chunk(ci, idxb, rows0, rows1, wbuf, sem):
            coff = ci * CH
            for c in range(8):
                pltpu.make_async_copy(sp.at[0].at[idxb.at[c]], rows0.at[c], sem).wait()
                pltpu.make_async_copy(sp.at[1].at[idxb.at[c]], rows1.at[c], sem).wait()

            @pl.loop(0, NG)
            def _g(g):
                g16 = coff + g * 16
                acc0 = jnp.zeros((16,), jnp.float32)
                acc1 = jnp.zeros((16,), jnp.float32)
                for c in range(8):
                    wv = wbuf[c, pl.ds(g * 16, 16)]
                    acc0 = acc0 + wv * rows0[c, pl.ds(g * 16, 16)]
                    acc1 = acc1 + wv * rows1[c, pl.ds(g * 16, 16)]
                lvlout[0, pl.ds(g16, 16)] = acc0
                lvlout[1, pl.ds(g16, 16)] = acc1

        @pl.loop(0, nch)
        def _seq(ci):
            compute_chunk(ci, idxA, r0A, r1A, semA)
            weights_chunk(ci, r0B)
            mac_chunk(ci, idxA, r0A, r1A, r0B, semA)

        pltpu.sync_copy(lvlout, out.at[pl.ds(lvl * 2, 2), pl.ds(wbase, spw)])
        plsc.subcore_barrier()

    for lvl in range(3):
        res = 16 << lvl

        def dense_idx(li, res=res):
            s0 = li[0] + li[1] * res + li[2] * (res * res)
            for c in range(8):
                cc = (c & 1) + ((c >> 1) & 1) * res + ((c >> 2) & 1) * (res * res)
                yield c, s0 + cc

        level_pass(lvl, res ** 3, dense_idx)

    @pl.loop(3, NUM_LEVELS)
    def _lvl(lvl):
        def hash_idx(li):
            hx = (li[0], li[0] + 1)
            hy0 = li[1] * P1
            hy = (hy0, hy0 + P1)
            hz0 = li[2] * P2
            hz = (hz0, hz0 + P2)
            for c in range(8):
                h = hx[c & 1] ^ hy[(c >> 1) & 1] ^ hz[(c >> 2) & 1]
                yield c, h & HASH_MASK

        level_pass(lvl, None, hash_idx)


def kernel(inputs, embeddings):
    b = inputs.shape[0]
    assert b % (NW * CH * 2) == 0
    spw = b // NW
    nch = spw // CH
    mesh = plsc.VectorSubcoreMesh(
        core_axis_name="c", subcore_axis_name="s", num_cores=NC, num_subcores=NS
    )
    fn = pl.kernel(
        functools.partial(_encode_body, spw, nch),
        out_type=jax.ShapeDtypeStruct((NUM_LEVELS * LEVEL_DIM, b), jnp.float32),
        mesh=mesh,
        compiler_params=pltpu.CompilerParams(use_tc_tiling_on_sc=False),
        scratch_types=[
            pltpu.VMEM((3, spw), jnp.float32),
            pltpu.VMEM((2, spw), jnp.float32),
            pltpu.VMEM((8, CH), jnp.int32),
            pltpu.VMEM((8, CH), jnp.int32),
            pltpu.VMEM((8, CH), jnp.float32),
            pltpu.VMEM((8, CH), jnp.float32),
            pltpu.VMEM((8, CH), jnp.float32),
            pltpu.VMEM((8, CH), jnp.float32),
            pltpu.VMEM_SHARED((2, HASH_N), jnp.float32),
            pltpu.SemaphoreType.DMA,
            pltpu.SemaphoreType.DMA,
        ],
    )
    embp = embeddings.T
    return fn(inputs.T, embp[0], embp[1]).T


# speedup vs baseline: 3.4258x; 1.0620x over previous
import functools

import numpy as np
import jax
import jax.numpy as jnp
from jax import lax
from jax.experimental import pallas as pl
from jax.experimental.pallas import tpu as pltpu
from jax.experimental.pallas import tpu_sc as plsc

NUM_LEVELS = 16
LEVEL_DIM = 2
P1 = int(np.uint32(2654435761).view(np.int32))
P2 = 805459861
HASH_MASK = (1 << 19) - 1
HASH_N = 1 << 19

_OFFS = [0, 4096, 36864, 299008]
for _ in range(13):
    _OFFS.append(_OFFS[-1] + HASH_N)

NC, NS = 2, 16
NW = NC * NS
CH = 128
NG = CH // 16


def _encode_body(
    spw, nch, inp, emb0, emb1, out,
    xyz, lvlout, idxA, idxB, r0A, r1A, r0B, r1B, wbA, wbB, sp, semA, semB,
):
    cid = lax.axis_index("c")
    sid = lax.axis_index("s")
    wid = sid * NC + cid
    wbase = wid * spw

    def corner_w(wa, wbv, c):
        w = wbv[0] if (c & 1) else wa[0]
        w = w * (wbv[1] if (c >> 1) & 1 else wa[1])
        w = w * (wbv[2] if (c >> 2) & 1 else wa[2])
        return w

    def smooth(g16, scale_f):
        li, wa, wbv = [], [], []
        for d in range(3):
            x01 = (xyz[d, pl.ds(g16, 16)] + 1.0) * 0.5
            pos = x01 * scale_f
            lid = pos.astype(jnp.int32)
            t = pos - lid.astype(jnp.float32)
            w = (t * t) * (3.0 - (t + t))
            li.append(lid)
            wbv.append(w)
            wa.append(1.0 - w)
        return li, wa, wbv

    pltpu.sync_copy(inp.at[:, pl.ds(wbase, spw)], xyz)

    def level_pass(lvl, nrows, idx_of_group):
        @pl.when(sid == 0)
        def _():
            off = _OFFS[3] + (lvl - 3) * HASH_N if nrows is None else _OFFS[lvl]
            n = HASH_N if nrows is None else nrows
            pltpu.sync_copy(emb0.at[pl.ds(off, n)], sp.at[0, pl.ds(0, n)])
            pltpu.sync_copy(emb1.at[pl.ds(off, n)], sp.at[1, pl.ds(0, n)])

        plsc.subcore_barrier()

        res = 16 << lvl
        scale_f = (res - 1).astype(jnp.float32) if nrows is None else float(res - 1)

        def compute_chunk(ci, idxb, rows0, rows1, sem):
            coff = ci * CH

            @pl.loop(0, NG)
            def _g(g):
                g16 = coff + g * 16
                li, _, _ = smooth(g16, scale_f)
                for c, iv in idx_of_group(li):
                    idxb[c, pl.ds(g * 16, 16)] = iv

            for c in range(8):
                pltpu.async_copy(sp.at[0].at[idxb.at[c]], rows0.at[c], sem)
                pltpu.async_copy(sp.at[1].at[idxb.at[c]], rows1.at[c], sem)

        def weights_chunk(ci, wbuf):
            coff = ci * CH

            @pl.loop(0, NG)
            def _g(g):
                g16 = coff + g * 16
                _, wa, wbv = smooth(g16, scale_f)
                for c in range(8):
                    wbuf[c, pl.ds(g * 16, 16)] = corner_w(wa, wbv, c)

        def mac_chunk(ci, idxb, rows0, rows1, wbuf, sem):
            coff = ci * CH
            for c in range(8):
                pltpu.make_async_copy(sp.at[0].at[idxb.at[c]], rows0.at[c], sem).wait()
                pltpu.make_async_copy(sp.at[1].at[idxb.at[c]], rows1.at[c], sem).wait()

            @pl.loop(0, NG)
            def _g(g):
                g16 = coff + g * 16
                acc0 = jnp.zeros((16,), jnp.float32)
                acc1 = jnp.zeros((16,), jnp.float32)
                for c in range(8):
                    wv = wbuf[c, pl.ds(g * 16, 16)]
                    acc0 = acc0 + wv * rows0[c, pl.ds(g * 16, 16)]
                    acc1 = acc1 + wv * rows1[c, pl.ds(g * 16, 16)]
                lvlout[0, pl.ds(g16, 16)] = acc0
                lvlout[1, pl.ds(g16, 16)] = acc1

        @pl.loop(0, nch // 2)
        def _seq(p):
            even = p * 2
            compute_chunk(even, idxA, r0A, r1A, semA)
            compute_chunk(even + 1, idxB, r0B, r1B, semB)
            weights_chunk(even, wbA)
            mac_chunk(even, idxA, r0A, r1A, wbA, semA)
            weights_chunk(even + 1, wbB)
            mac_chunk(even + 1, idxB, r0B, r1B, wbB, semB)

        pltpu.sync_copy(lvlout, out.at[pl.ds(lvl * 2, 2), pl.ds(wbase, spw)])
        plsc.subcore_barrier()

    for lvl in range(3):
        res = 16 << lvl

        def dense_idx(li, res=res):
            s0 = li[0] + li[1] * res + li[2] * (res * res)
            for c in range(8):
                cc = (c & 1) + ((c >> 1) & 1) * res + ((c >> 2) & 1) * (res * res)
                yield c, s0 + cc

        level_pass(lvl, res ** 3, dense_idx)

    @pl.loop(3, NUM_LEVELS)
    def _lvl(lvl):
        def hash_idx(li):
            hx = (li[0], li[0] + 1)
            hy0 = li[1] * P1
            hy = (hy0, hy0 + P1)
            hz0 = li[2] * P2
            hz = (hz0, hz0 + P2)
            for c in range(8):
                h = hx[c & 1] ^ hy[(c >> 1) & 1] ^ hz[(c >> 2) & 1]
                yield c, h & HASH_MASK

        level_pass(lvl, None, hash_idx)


def kernel(inputs, embeddings):
    b = inputs.shape[0]
    assert b % (NW * CH * 2) == 0
    spw = b // NW
    nch = spw // CH
    mesh = plsc.VectorSubcoreMesh(
        core_axis_name="c", subcore_axis_name="s", num_cores=NC, num_subcores=NS
    )
    fn = pl.kernel(
        functools.partial(_encode_body, spw, nch),
        out_type=jax.ShapeDtypeStruct((NUM_LEVELS * LEVEL_DIM, b), jnp.float32),
        mesh=mesh,
        compiler_params=pltpu.CompilerParams(use_tc_tiling_on_sc=False),
        scratch_types=[
            pltpu.VMEM((3, spw), jnp.float32),
            pltpu.VMEM((2, spw), jnp.float32),
            pltpu.VMEM((8, CH), jnp.int32),
            pltpu.VMEM((8, CH), jnp.int32),
            pltpu.VMEM((8, CH), jnp.float32),
            pltpu.VMEM((8, CH), jnp.float32),
            pltpu.VMEM((8, CH), jnp.float32),
            pltpu.VMEM((8, CH), jnp.float32),
            pltpu.VMEM((8, CH), jnp.float32),
            pltpu.VMEM((8, CH), jnp.float32),
            pltpu.VMEM_SHARED((2, HASH_N), jnp.float32),
            pltpu.SemaphoreType.DMA,
            pltpu.SemaphoreType.DMA,
        ],
    )
    embp = embeddings.T
    return fn(inputs.T, embp[0], embp[1]).T


# speedup vs baseline: 3.6156x; 1.0554x over previous
import functools

import numpy as np
import jax
import jax.numpy as jnp
from jax import lax
from jax.experimental import pallas as pl
from jax.experimental.pallas import tpu as pltpu
from jax.experimental.pallas import tpu_sc as plsc

NUM_LEVELS = 16
LEVEL_DIM = 2
P1 = int(np.uint32(2654435761).view(np.int32))
P2 = 805459861
HASH_MASK = (1 << 19) - 1
HASH_N = 1 << 19

_OFFS = [0, 4096, 36864, 299008]
for _ in range(13):
    _OFFS.append(_OFFS[-1] + HASH_N)

NC, NS = 2, 16
NW = NC * NS
CH = 128
NG = CH // 16


def _encode_body(
    spw, nch, inp, emb0, emb1, out,
    xyz, lvlout, idxA, idxB, r0A, r1A, r0B, r1B, wbA, wbB, sp, semA, semB,
):
    cid = lax.axis_index("c")
    sid = lax.axis_index("s")
    wid = sid * NC + cid
    wbase = wid * spw

    def corner_w(wa, wbv, c):
        w = wbv[0] if (c & 1) else wa[0]
        w = w * (wbv[1] if (c >> 1) & 1 else wa[1])
        w = w * (wbv[2] if (c >> 2) & 1 else wa[2])
        return w

    def smooth(g16, scale_f):
        li, wa, wbv = [], [], []
        for d in range(3):
            x01 = (xyz[d, pl.ds(g16, 16)] + 1.0) * 0.5
            pos = x01 * scale_f
            lid = pos.astype(jnp.int32)
            t = pos - lid.astype(jnp.float32)
            w = (t * t) * (3.0 - (t + t))
            li.append(lid)
            wbv.append(w)
            wa.append(1.0 - w)
        return li, wa, wbv

    pltpu.sync_copy(inp.at[:, pl.ds(wbase, spw)], xyz)

    def level_pass(lvl, nrows, idx_of_group):
        @pl.when(sid == 0)
        def _():
            off = _OFFS[3] + (lvl - 3) * HASH_N if nrows is None else _OFFS[lvl]
            n = HASH_N if nrows is None else nrows
            pltpu.sync_copy(emb0.at[pl.ds(off, n)], sp.at[0, pl.ds(0, n)])
            pltpu.sync_copy(emb1.at[pl.ds(off, n)], sp.at[1, pl.ds(0, n)])

        plsc.subcore_barrier()

        res = 16 << lvl
        scale_f = (res - 1).astype(jnp.float32) if nrows is None else float(res - 1)

        def compute_chunk(ci, idxb, wbuf, rows0, rows1, sem):
            coff = ci * CH

            @pl.loop(0, NG)
            def _g(g):
                g16 = coff + g * 16
                li, wa, wbv = smooth(g16, scale_f)
                for c, iv in idx_of_group(li):
                    idxb[c, pl.ds(g * 16, 16)] = iv
                wxy = (wa[0] * wa[1], wbv[0] * wa[1], wa[0] * wbv[1], wbv[0] * wbv[1])
                for c in range(8):
                    wz = wbv[2] if (c >> 2) & 1 else wa[2]
                    wbuf[c, pl.ds(g * 16, 16)] = wxy[c & 3] * wz

            for c in range(8):
                pltpu.async_copy(sp.at[0].at[idxb.at[c]], rows0.at[c], sem)
                pltpu.async_copy(sp.at[1].at[idxb.at[c]], rows1.at[c], sem)

        def mac_chunk(ci, idxb, rows0, rows1, wbuf, sem):
            coff = ci * CH
            for c in range(8):
                pltpu.make_async_copy(sp.at[0].at[idxb.at[c]], rows0.at[c], sem).wait()
                pltpu.make_async_copy(sp.at[1].at[idxb.at[c]], rows1.at[c], sem).wait()

            @pl.loop(0, NG)
            def _g(g):
                g16 = coff + g * 16
                acc0 = jnp.zeros((16,), jnp.float32)
                acc1 = jnp.zeros((16,), jnp.float32)
                for c in range(8):
                    wv = wbuf[c, pl.ds(g * 16, 16)]
                    acc0 = acc0 + wv * rows0[c, pl.ds(g * 16, 16)]
                    acc1 = acc1 + wv * rows1[c, pl.ds(g * 16, 16)]
                lvlout[0, pl.ds(g16, 16)] = acc0
                lvlout[1, pl.ds(g16, 16)] = acc1

        @pl.loop(0, nch // 2)
        def _seq(p):
            even = p * 2
            compute_chunk(even, idxA, wbA, r0A, r1A, semA)
            compute_chunk(even + 1, idxB, wbB, r0B, r1B, semB)
            mac_chunk(even, idxA, r0A, r1A, wbA, semA)
            mac_chunk(even + 1, idxB, r0B, r1B, wbB, semB)

        pltpu.sync_copy(lvlout, out.at[pl.ds(lvl * 2, 2), pl.ds(wbase, spw)])
        plsc.subcore_barrier()

    for lvl in range(3):
        res = 16 << lvl

        def dense_idx(li, res=res):
            s0 = li[0] + li[1] * res + li[2] * (res * res)
            for c in range(8):
                cc = (c & 1) + ((c >> 1) & 1) * res + ((c >> 2) & 1) * (res * res)
                yield c, s0 + cc

        level_pass(lvl, res ** 3, dense_idx)

    @pl.loop(3, NUM_LEVELS)
    def _lvl(lvl):
        def hash_idx(li):
            hx = (li[0], li[0] + 1)
            hy0 = li[1] * P1
            hy = (hy0, hy0 + P1)
            hz0 = li[2] * P2
            hz = (hz0, hz0 + P2)
            for c in range(8):
                h = hx[c & 1] ^ hy[(c >> 1) & 1] ^ hz[(c >> 2) & 1]
                yield c, h & HASH_MASK

        level_pass(lvl, None, hash_idx)


def kernel(inputs, embeddings):
    b = inputs.shape[0]
    assert b % (NW * CH * 2) == 0
    spw = b // NW
    nch = spw // CH
    mesh = plsc.VectorSubcoreMesh(
        core_axis_name="c", subcore_axis_name="s", num_cores=NC, num_subcores=NS
    )
    fn = pl.kernel(
        functools.partial(_encode_body, spw, nch),
        out_type=jax.ShapeDtypeStruct((NUM_LEVELS * LEVEL_DIM, b), jnp.float32),
        mesh=mesh,
        compiler_params=pltpu.CompilerParams(use_tc_tiling_on_sc=False),
        scratch_types=[
            pltpu.VMEM((3, spw), jnp.float32),
            pltpu.VMEM((2, spw), jnp.float32),
            pltpu.VMEM((8, CH), jnp.int32),
            pltpu.VMEM((8, CH), jnp.int32),
            pltpu.VMEM((8, CH), jnp.float32),
            pltpu.VMEM((8, CH), jnp.float32),
            pltpu.VMEM((8, CH), jnp.float32),
            pltpu.VMEM((8, CH), jnp.float32),
            pltpu.VMEM((8, CH), jnp.float32),
            pltpu.VMEM((8, CH), jnp.float32),
            pltpu.VMEM_SHARED((2, HASH_N), jnp.float32),
            pltpu.SemaphoreType.DMA,
            pltpu.SemaphoreType.DMA,
        ],
    )
    embp = embeddings.T
    return fn(inputs.T, embp[0], embp[1]).T


# speedup vs baseline: 3.9418x; 1.0902x over previous
import functools

import numpy as np
import jax
import jax.numpy as jnp
from jax import lax
from jax.experimental import pallas as pl
from jax.experimental.pallas import tpu as pltpu
from jax.experimental.pallas import tpu_sc as plsc

NUM_LEVELS = 16
LEVEL_DIM = 2
P1 = int(np.uint32(2654435761).view(np.int32))
P2 = 805459861
HASH_MASK = (1 << 19) - 1
HASH_N = 1 << 19

_OFFS = [0, 4096, 36864, 299008]
for _ in range(13):
    _OFFS.append(_OFFS[-1] + HASH_N)

NC, NS = 2, 16
NW = NC * NS
CH = 128
NG = CH // 16


def _encode_body(
    spw, nch, inp, emb0, emb1, out,
    xyz, lvlout, idxA, idxB, idxC, idxD, r0A, r1A, r0B, r1B, r0C, r1C, r0D, r1D,
    wbA, wbB, wbC, wbD, sp, semA, semB, semC, semD,
):
    cid = lax.axis_index("c")
    sid = lax.axis_index("s")
    wid = sid * NC + cid
    wbase = wid * spw

    def corner_w(wa, wbv, c):
        w = wbv[0] if (c & 1) else wa[0]
        w = w * (wbv[1] if (c >> 1) & 1 else wa[1])
        w = w * (wbv[2] if (c >> 2) & 1 else wa[2])
        return w

    def smooth(g16, scale_f):
        li, wa, wbv = [], [], []
        for d in range(3):
            x01 = (xyz[d, pl.ds(g16, 16)] + 1.0) * 0.5
            pos = x01 * scale_f
            lid = pos.astype(jnp.int32)
            t = pos - lid.astype(jnp.float32)
            w = (t * t) * (3.0 - (t + t))
            li.append(lid)
            wbv.append(w)
            wa.append(1.0 - w)
        return li, wa, wbv

    pltpu.sync_copy(inp.at[:, pl.ds(wbase, spw)], xyz)

    def level_pass(lvl, nrows, idx_of_group):
        @pl.when(sid == 0)
        def _():
            off = _OFFS[3] + (lvl - 3) * HASH_N if nrows is None else _OFFS[lvl]
            n = HASH_N if nrows is None else nrows
            pltpu.sync_copy(emb0.at[pl.ds(off, n)], sp.at[0, pl.ds(0, n)])
            pltpu.sync_copy(emb1.at[pl.ds(off, n)], sp.at[1, pl.ds(0, n)])

        plsc.subcore_barrier()

        res = 16 << lvl
        scale_f = (res - 1).astype(jnp.float32) if nrows is None else float(res - 1)

        def compute_chunk(ci, idxb, wbuf, rows0, rows1, sem):
            coff = ci * CH

            @pl.loop(0, NG)
            def _g(g):
                g16 = coff + g * 16
                li, wa, wbv = smooth(g16, scale_f)
                for c, iv in idx_of_group(li):
                    idxb[c, pl.ds(g * 16, 16)] = iv
                wxy = (wa[0] * wa[1], wbv[0] * wa[1], wa[0] * wbv[1], wbv[0] * wbv[1])
                for c in range(8):
                    wz = wbv[2] if (c >> 2) & 1 else wa[2]
                    wbuf[c, pl.ds(g * 16, 16)] = wxy[c & 3] * wz

            for c in range(8):
                pltpu.async_copy(sp.at[0].at[idxb.at[c]], rows0.at[c], sem)
                pltpu.async_copy(sp.at[1].at[idxb.at[c]], rows1.at[c], sem)

        def mac_chunk(ci, idxb, rows0, rows1, wbuf, sem):
            coff = ci * CH
            for c in range(8):
                pltpu.make_async_copy(sp.at[0].at[idxb.at[c]], rows0.at[c], sem).wait()
                pltpu.make_async_copy(sp.at[1].at[idxb.at[c]], rows1.at[c], sem).wait()

            @pl.loop(0, NG)
            def _g(g):
                g16 = coff + g * 16
                acc0 = jnp.zeros((16,), jnp.float32)
                acc1 = jnp.zeros((16,), jnp.float32)
                for c in range(8):
                    wv = wbuf[c, pl.ds(g * 16, 16)]
                    acc0 = acc0 + wv * rows0[c, pl.ds(g * 16, 16)]
                    acc1 = acc1 + wv * rows1[c, pl.ds(g * 16, 16)]
                lvlout[0, pl.ds(g16, 16)] = acc0
                lvlout[1, pl.ds(g16, 16)] = acc1

        @pl.loop(0, nch // 4)
        def _seq(p):
            c0 = p * 4
            compute_chunk(c0, idxA, wbA, r0A, r1A, semA)
            compute_chunk(c0 + 1, idxB, wbB, r0B, r1B, semB)
            compute_chunk(c0 + 2, idxC, wbC, r0C, r1C, semC)
            compute_chunk(c0 + 3, idxD, wbD, r0D, r1D, semD)
            mac_chunk(c0, idxA, r0A, r1A, wbA, semA)
            mac_chunk(c0 + 1, idxB, r0B, r1B, wbB, semB)
            mac_chunk(c0 + 2, idxC, r0C, r1C, wbC, semC)
            mac_chunk(c0 + 3, idxD, r0D, r1D, wbD, semD)

        pltpu.sync_copy(lvlout, out.at[pl.ds(lvl * 2, 2), pl.ds(wbase, spw)])
        plsc.subcore_barrier()

    for lvl in range(3):
        res = 16 << lvl

        def dense_idx(li, res=res):
            s0 = li[0] + li[1] * res + li[2] * (res * res)
            for c in range(8):
                cc = (c & 1) + ((c >> 1) & 1) * res + ((c >> 2) & 1) * (res * res)
                yield c, s0 + cc

        level_pass(lvl, res ** 3, dense_idx)

    @pl.loop(3, NUM_LEVELS)
    def _lvl(lvl):
        def hash_idx(li):
            hx = (li[0], li[0] + 1)
            hy0 = li[1] * P1
            hy = (hy0, hy0 + P1)
            hz0 = li[2] * P2
            hz = (hz0, hz0 + P2)
            for c in range(8):
                h = hx[c & 1] ^ hy[(c >> 1) & 1] ^ hz[(c >> 2) & 1]
                yield c, h & HASH_MASK

        level_pass(lvl, None, hash_idx)


def kernel(inputs, embeddings):
    b = inputs.shape[0]
    assert b % (NW * CH * 2) == 0
    spw = b // NW
    nch = spw // CH
    mesh = plsc.VectorSubcoreMesh(
        core_axis_name="c", subcore_axis_name="s", num_cores=NC, num_subcores=NS
    )
    fn = pl.kernel(
        functools.partial(_encode_body, spw, nch),
        out_type=jax.ShapeDtypeStruct((NUM_LEVELS * LEVEL_DIM, b), jnp.float32),
        mesh=mesh,
        compiler_params=pltpu.CompilerParams(use_tc_tiling_on_sc=False),
        scratch_types=[
            pltpu.VMEM((3, spw), jnp.float32),
            pltpu.VMEM((2, spw), jnp.float32),
            pltpu.VMEM((8, CH), jnp.int32),
            pltpu.VMEM((8, CH), jnp.int32),
            pltpu.VMEM((8, CH), jnp.int32),
            pltpu.VMEM((8, CH), jnp.int32),
            pltpu.VMEM((8, CH), jnp.float32),
            pltpu.VMEM((8, CH), jnp.float32),
            pltpu.VMEM((8, CH), jnp.float32),
            pltpu.VMEM((8, CH), jnp.float32),
            pltpu.VMEM((8, CH), jnp.float32),
            pltpu.VMEM((8, CH), jnp.float32),
            pltpu.VMEM((8, CH), jnp.float32),
            pltpu.VMEM((8, CH), jnp.float32),
            pltpu.VMEM((8, CH), jnp.float32),
            pltpu.VMEM((8, CH), jnp.float32),
            pltpu.VMEM((8, CH), jnp.float32),
            pltpu.VMEM((8, CH), jnp.float32),
            pltpu.VMEM_SHARED((2, HASH_N), jnp.float32),
            pltpu.SemaphoreType.DMA,
            pltpu.SemaphoreType.DMA,
            pltpu.SemaphoreType.DMA,
            pltpu.SemaphoreType.DMA,
        ],
    )
    embp = embeddings.T
    return fn(inputs.T, embp[0], embp[1]).T


# speedup vs baseline: 3.9785x; 1.0093x over previous
import functools

import numpy as np
import jax
import jax.numpy as jnp
from jax import lax
from jax.experimental import pallas as pl
from jax.experimental.pallas import tpu as pltpu
from jax.experimental.pallas import tpu_sc as plsc

NUM_LEVELS = 16
LEVEL_DIM = 2
P1 = int(np.uint32(2654435761).view(np.int32))
P2 = 805459861
HASH_MASK = (1 << 19) - 1
HASH_N = 1 << 19

_OFFS = [0, 4096, 36864, 299008]
for _ in range(13):
    _OFFS.append(_OFFS[-1] + HASH_N)

NC, NS = 2, 16
NW = NC * NS
CH = 128
NG = CH // 16


def _encode_body(
    spw, nch, inp, emb0, emb1, out,
    xyz, lvlout, idxA, idxB, idxC, idxD, r0A, r1A, r0B, r1B, r0C, r1C, r0D, r1D,
    wbA, wbB, wbC, wbD, sp, semA, semB, semC, semD,
):
    cid = lax.axis_index("c")
    sid = lax.axis_index("s")
    wid = sid * NC + cid
    wbase = wid * spw

    def corner_w(wa, wbv, c):
        w = wbv[0] if (c & 1) else wa[0]
        w = w * (wbv[1] if (c >> 1) & 1 else wa[1])
        w = w * (wbv[2] if (c >> 2) & 1 else wa[2])
        return w

    def smooth(g16, scale_f):
        li, wa, wbv = [], [], []
        for d in range(3):
            x01 = (xyz[d, pl.ds(g16, 16)] + 1.0) * 0.5
            pos = x01 * scale_f
            lid = pos.astype(jnp.int32)
            t = pos - lid.astype(jnp.float32)
            w = (t * t) * (3.0 - (t + t))
            li.append(lid)
            wbv.append(w)
            wa.append(1.0 - w)
        return li, wa, wbv

    pltpu.sync_copy(inp.at[:, pl.ds(wbase, spw)], xyz)

    def level_pass(lvl, nrows, idx_of_group):
        off = _OFFS[3] + (lvl - 3) * HASH_N if nrows is None else _OFFS[lvl]
        n = HASH_N if nrows is None else nrows
        nsub = n // NS
        s0 = sid * nsub
        pltpu.sync_copy(emb0.at[pl.ds(off + s0, nsub)], sp.at[0, pl.ds(s0, nsub)])
        pltpu.sync_copy(emb1.at[pl.ds(off + s0, nsub)], sp.at[1, pl.ds(s0, nsub)])

        plsc.subcore_barrier()

        res = 16 << lvl
        scale_f = (res - 1).astype(jnp.float32) if nrows is None else float(res - 1)

        def compute_chunk(ci, idxb, wbuf, rows0, rows1, sem):
            coff = ci * CH

            @pl.loop(0, NG)
            def _g(g):
                g16 = coff + g * 16
                li, wa, wbv = smooth(g16, scale_f)
                for c, iv in idx_of_group(li):
                    idxb[c, pl.ds(g * 16, 16)] = iv
                wxy = (wa[0] * wa[1], wbv[0] * wa[1], wa[0] * wbv[1], wbv[0] * wbv[1])
                for c in range(8):
                    wz = wbv[2] if (c >> 2) & 1 else wa[2]
                    wbuf[c, pl.ds(g * 16, 16)] = wxy[c & 3] * wz

            for c in range(8):
                pltpu.async_copy(sp.at[0].at[idxb.at[c]], rows0.at[c], sem)
                pltpu.async_copy(sp.at[1].at[idxb.at[c]], rows1.at[c], sem)

        def mac_chunk(ci, idxb, rows0, rows1, wbuf, sem):
            coff = ci * CH
            for c in range(8):
                pltpu.make_async_copy(sp.at[0].at[idxb.at[c]], rows0.at[c], sem).wait()
                pltpu.make_async_copy(sp.at[1].at[idxb.at[c]], rows1.at[c], sem).wait()

            @pl.loop(0, NG)
            def _g(g):
                g16 = coff + g * 16
                acc0 = jnp.zeros((16,), jnp.float32)
                acc1 = jnp.zeros((16,), jnp.float32)
                for c in range(8):
                    wv = wbuf[c, pl.ds(g * 16, 16)]
                    acc0 = acc0 + wv * rows0[c, pl.ds(g * 16, 16)]
                    acc1 = acc1 + wv * rows1[c, pl.ds(g * 16, 16)]
                lvlout[0, pl.ds(g16, 16)] = acc0
                lvlout[1, pl.ds(g16, 16)] = acc1

        @pl.loop(0, nch // 4)
        def _seq(p):
            c0 = p * 4
            compute_chunk(c0, idxA, wbA, r0A, r1A, semA)
            compute_chunk(c0 + 1, idxB, wbB, r0B, r1B, semB)
            compute_chunk(c0 + 2, idxC, wbC, r0C, r1C, semC)
            compute_chunk(c0 + 3, idxD, wbD, r0D, r1D, semD)
            mac_chunk(c0, idxA, r0A, r1A, wbA, semA)
            mac_chunk(c0 + 1, idxB, r0B, r1B, wbB, semB)
            mac_chunk(c0 + 2, idxC, r0C, r1C, wbC, semC)
            mac_chunk(c0 + 3, idxD, r0D, r1D, wbD, semD)

        pltpu.sync_copy(lvlout, out.at[pl.ds(lvl * 2, 2), pl.ds(wbase, spw)])
        plsc.subcore_barrier()

    for lvl in range(3):
        res = 16 << lvl

        def dense_idx(li, res=res):
            s0 = li[0] + li[1] * res + li[2] * (res * res)
            for c in range(8):
                cc = (c & 1) + ((c >> 1) & 1) * res + ((c >> 2) & 1) * (res * res)
                yield c, s0 + cc

        level_pass(lvl, res ** 3, dense_idx)

    @pl.loop(3, NUM_LEVELS)
    def _lvl(lvl):
        def hash_idx(li):
            hx = (li[0], li[0] + 1)
            hy0 = li[1] * P1
            hy = (hy0, hy0 + P1)
            hz0 = li[2] * P2
            hz = (hz0, hz0 + P2)
            for c in range(8):
                h = hx[c & 1] ^ hy[(c >> 1) & 1] ^ hz[(c >> 2) & 1]
                yield c, h & HASH_MASK

        level_pass(lvl, None, hash_idx)


def kernel(inputs, embeddings):
    b = inputs.shape[0]
    assert b % (NW * CH * 2) == 0
    spw = b // NW
    nch = spw // CH
    mesh = plsc.VectorSubcoreMesh(
        core_axis_name="c", subcore_axis_name="s", num_cores=NC, num_subcores=NS
    )
    fn = pl.kernel(
        functools.partial(_encode_body, spw, nch),
        out_type=jax.ShapeDtypeStruct((NUM_LEVELS * LEVEL_DIM, b), jnp.float32),
        mesh=mesh,
        compiler_params=pltpu.CompilerParams(use_tc_tiling_on_sc=False),
        scratch_types=[
            pltpu.VMEM((3, spw), jnp.float32),
            pltpu.VMEM((2, spw), jnp.float32),
            pltpu.VMEM((8, CH), jnp.int32),
            pltpu.VMEM((8, CH), jnp.int32),
            pltpu.VMEM((8, CH), jnp.int32),
            pltpu.VMEM((8, CH), jnp.int32),
            pltpu.VMEM((8, CH), jnp.float32),
            pltpu.VMEM((8, CH), jnp.float32),
            pltpu.VMEM((8, CH), jnp.float32),
            pltpu.VMEM((8, CH), jnp.float32),
            pltpu.VMEM((8, CH), jnp.float32),
            pltpu.VMEM((8, CH), jnp.float32),
            pltpu.VMEM((8, CH), jnp.float32),
            pltpu.VMEM((8, CH), jnp.float32),
            pltpu.VMEM((8, CH), jnp.float32),
            pltpu.VMEM((8, CH), jnp.float32),
            pltpu.VMEM((8, CH), jnp.float32),
            pltpu.VMEM((8, CH), jnp.float32),
            pltpu.VMEM_SHARED((2, HASH_N), jnp.float32),
            pltpu.SemaphoreType.DMA,
            pltpu.SemaphoreType.DMA,
            pltpu.SemaphoreType.DMA,
            pltpu.SemaphoreType.DMA,
        ],
    )
    embp = embeddings.T
    return fn(inputs.T, embp[0], embp[1]).T
